# baseline (speedup 1.0000x reference)
import jax
import jax.numpy as jnp
from jax import lax
from jax.experimental import pallas as pl
from jax.experimental.pallas import tpu as pltpu
from jax.experimental.pallas import tpu_sc as plsc

N = 10000
NPAD = 10240
D = 128
CH = 128
NW = 32
ROWS_PER_TILE = NPAD // 16
PAD_IDX = N

F32 = jnp.float32


def _pad_edges(ep):
    q = 8 * NW * CH
    return ((ep + q - 1) // q) * q


_BM = 512


def _mm2_same_x(x, Wl, bl, Wr, br):
    def body(x_ref, wl_ref, bl_ref, wr_ref, br_ref, o1_ref, o2_ref):
        xb = x_ref[...]
        o1_ref[...] = jnp.dot(xb, wl_ref[...], preferred_element_type=F32) + bl_ref[...]
        o2_ref[...] = jnp.dot(xb, wr_ref[...], preferred_element_type=F32) + br_ref[...]

    grid = (NPAD // _BM,)
    return pl.pallas_call(
        body,
        grid=grid,
        in_specs=[
            pl.BlockSpec((_BM, D), lambda i: (i, 0)),
            pl.BlockSpec((D, D), lambda i: (0, 0)),
            pl.BlockSpec((1, D), lambda i: (0, 0)),
            pl.BlockSpec((D, D), lambda i: (0, 0)),
            pl.BlockSpec((1, D), lambda i: (0, 0)),
        ],
        out_specs=[
            pl.BlockSpec((_BM, D), lambda i: (i, 0)),
            pl.BlockSpec((_BM, D), lambda i: (i, 0)),
        ],
        out_shape=[
            jax.ShapeDtypeStruct((NPAD, D), F32),
            jax.ShapeDtypeStruct((NPAD, D), F32),
        ],
    )(x, Wl, bl.reshape(1, D), Wr, br.reshape(1, D))


def _mm2_two_x(xl, Wl, bl, xr, Wr, br):
    def body(xl_ref, wl_ref, bl_ref, xr_ref, wr_ref, br_ref, o1_ref, o2_ref):
        o1_ref[...] = jnp.dot(xl_ref[...], wl_ref[...], preferred_element_type=F32) + bl_ref[...]
        o2_ref[...] = jnp.dot(xr_ref[...], wr_ref[...], preferred_element_type=F32) + br_ref[...]

    grid = (NPAD // _BM,)
    return pl.pallas_call(
        body,
        grid=grid,
        in_specs=[
            pl.BlockSpec((_BM, D), lambda i: (i, 0)),
            pl.BlockSpec((D, D), lambda i: (0, 0)),
            pl.BlockSpec((1, D), lambda i: (0, 0)),
            pl.BlockSpec((_BM, D), lambda i: (i, 0)),
            pl.BlockSpec((D, D), lambda i: (0, 0)),
            pl.BlockSpec((1, D), lambda i: (0, 0)),
        ],
        out_specs=[
            pl.BlockSpec((_BM, D), lambda i: (i, 0)),
            pl.BlockSpec((_BM, D), lambda i: (i, 0)),
        ],
        out_shape=[
            jax.ShapeDtypeStruct((NPAD, D), F32),
            jax.ShapeDtypeStruct((NPAD, D), F32),
        ],
    )(xl, Wl, bl.reshape(1, D), xr, Wr, br.reshape(1, D))


def _combine_mm(p0, p1, b, W, c):
    def body(p0_ref, p1_ref, b_ref, w_ref, c_ref, o_ref):
        y = jax.nn.relu(p0_ref[...] + p1_ref[...] + b_ref[...])
        o_ref[...] = jnp.dot(y, w_ref[...], preferred_element_type=F32) + c_ref[...]

    grid = (NPAD // _BM,)
    return pl.pallas_call(
        body,
        grid=grid,
        in_specs=[
            pl.BlockSpec((_BM, D), lambda i: (i, 0)),
            pl.BlockSpec((_BM, D), lambda i: (i, 0)),
            pl.BlockSpec((1, D), lambda i: (0, 0)),
            pl.BlockSpec((D, D), lambda i: (0, 0)),
            pl.BlockSpec((1, D), lambda i: (0, 0)),
        ],
        out_specs=pl.BlockSpec((_BM, D), lambda i: (i, 0)),
        out_shape=jax.ShapeDtypeStruct((NPAD, D), F32),
    )(p0, p1, b.reshape(1, D), W, c.reshape(1, D))


def _make_pass_a(ep_pad, has_edge):
    ept = ep_pad // NW
    nchunks = ept // CH
    mesh = plsc.VectorSubcoreMesh(core_axis_name="c", subcore_axis_name="s")

    scratch = [
        pltpu.VMEM((nchunks, CH), jnp.int32),
        pltpu.VMEM((nchunks, CH), jnp.int32),
        pltpu.VMEM((CH, D), F32),
        pltpu.VMEM((CH, D), F32),
        pltpu.VMEM((CH, D), F32),
        pltpu.VMEM((CH, D), F32),
        pltpu.VMEM((D,), F32),
        pltpu.VMEM((nchunks, CH), F32),
        pltpu.VMEM((ROWS_PER_TILE,), F32),
        pltpu.VMEM_SHARED((NPAD,), F32),
        pltpu.SemaphoreType.DMA,
        pltpu.SemaphoreType.DMA,
        pltpu.SemaphoreType.DMA,
        pltpu.SemaphoreType.DMA,
    ]
    if has_edge:
        scratch += [
            pltpu.VMEM((4, CH), F32),
            pltpu.VMEM((4, CH), F32),
            pltpu.VMEM((4, D), F32),
            pltpu.SemaphoreType.DMA,
            pltpu.SemaphoreType.DMA,
        ]

    def body(*refs):
        if has_edge:
            (hlh, hrh, srch, dsth, elh, weh, atth, exh, denph,
             src_all, dst_all, rl0, rl1, rr0, rr1, attv, exbig, zb, densh,
             semL0, semL1, semR0, semR1, elv0, elv1, wev, semE0, semE1) = refs
        else:
            (hlh, hrh, srch, dsth, atth, exh, denph,
             src_all, dst_all, rl0, rl1, rr0, rr1, attv, exbig, zb, densh,
             semL0, semL1, semR0, semR1) = refs

        c = lax.axis_index("c")
        s = lax.axis_index("s")
        wid = s * 2 + c
        row0 = wid * nchunks

        rl = [rl0, rl1]
        rr = [rr0, rr1]
        semL = [semL0, semL1]
        semR = [semR0, semR1]
        if has_edge:
            elv = [elv0, elv1]
            semE = [semE0, semE1]

        zeros16 = jnp.zeros((16,), F32)
        for i in range(ROWS_PER_TILE // 16):
            zb[pl.ds(i * 16, 16)] = zeros16
        pltpu.sync_copy(zb, densh.at[pl.ds(s * ROWS_PER_TILE, ROWS_PER_TILE)])
        pltpu.sync_copy(atth, attv)
        if has_edge:
            pltpu.sync_copy(weh, wev)
        pltpu.sync_copy(srch.at[pl.ds(row0, nchunks)], src_all)
        pltpu.sync_copy(dsth.at[pl.ds(row0, nchunks)], dst_all)
        plsc.subcore_barrier()

        def issue(ci, b):
            pltpu.async_copy(hlh.at[src_all.at[ci]], rl[b], semL[b])
            pltpu.async_copy(hrh.at[dst_all.at[ci]], rr[b], semR[b])
            if has_edge:
                for j in range(4):
                    pltpu.async_copy(elh.at[j, pl.ds((row0 + ci) * CH, CH)],
                                     elv[b].at[j], semE[b])

        def wait(b):
            pltpu.make_async_copy(hlh.at[src_all.at[0]], rl[b], semL[b]).wait()
            pltpu.make_async_copy(hrh.at[dst_all.at[0]], rr[b], semR[b]).wait()
            if has_edge:
                for j in range(4):
                    pltpu.make_async_copy(elh.at[j, pl.ds(0, CH)],
                                          elv[b].at[j], semE[b]).wait()

        attr = [attv[pl.ds(v * 16, 16)] for v in range(8)]
        if has_edge:
            wer = [[wev[j, pl.ds(v * 16, 16)] for v in range(8)] for j in range(4)]
        lane = lax.iota(jnp.int32, 16)
        masks = [lane == k for k in range(16)]

        def compute(ci, b):
            def group(g):
                if has_edge:
                    elg = [elv[b][j, pl.ds(g * 16, 16)] for j in range(4)]
                z = zeros16
                for k in range(16):
                    e = g * 16 + k
                    acc = zeros16
                    for v in range(8):
                        sl = pl.ds(v * 16, 16)
                        t = rl[b][e, sl] + rr[b][e, sl]
                        if has_edge:
                            t = (t + elg[0][k] * wer[0][v] + elg[1][k] * wer[1][v]
                                 + elg[2][k] * wer[2][v] + elg[3][k] * wer[3][v])
                        u = 0.6 * t + 0.4 * jnp.abs(t)
                        acc = acc + attr[v] * u
                    z = jnp.where(masks[k], jnp.sum(acc), z)
                exbig[ci, pl.ds(g * 16, 16)] = jnp.exp(z)

            pl.loop(0, CH // 16)(group)

        issue(0, 0)

        def step(ci):
            issue(ci + 1, 1)
            wait(0)
            compute(ci, 0)

            @pl.when(ci + 2 < nchunks)
            def _():
                issue(ci + 2, 0)

            wait(1)
            compute(ci + 1, 1)

        pl.loop(0, nchunks, step=2)(step)

        pltpu.sync_copy(exbig, exh.at[pl.ds(row0, nchunks)])
        nb = (nchunks // 8) * 8

        def scat8(ci):
            for j in range(8):
                pltpu.async_copy(exbig.at[ci + j], densh.at[dst_all.at[ci + j]],
                                 semL0, add=True)
            for j in range(8):
                pltpu.make_async_copy(exbig.at[0], densh.at[dst_all.at[0]],
                                      semL0).wait()

        pl.loop(0, nb, step=8)(scat8)
        for t in range(nchunks - nb):
            pltpu.sync_copy(exbig.at[nb + t], densh.at[dst_all.at[nb + t]],
                            add=True)
        plsc.subcore_barrier()

        off = s * ROWS_PER_TILE
        pltpu.sync_copy(densh.at[pl.ds(off, ROWS_PER_TILE)], zb)
        pltpu.sync_copy(zb, denph.at[c, pl.ds(off, ROWS_PER_TILE)])

    out_type = (
        jax.ShapeDtypeStruct((ep_pad // CH, CH), F32),
        jax.ShapeDtypeStruct((2, NPAD), F32),
    )
    return pl.kernel(body, out_type=out_type, mesh=mesh, scratch_types=scratch,
                     compiler_params=pltpu.CompilerParams(needs_layout_passes=False))


def _make_pass_b(ep_pad):
    ept = ep_pad // NW
    nchunks = ept // CH
    mesh = plsc.VectorSubcoreMesh(core_axis_name="c", subcore_axis_name="s")
    HALF = ROWS_PER_TILE // 2

    scratch = [
        pltpu.VMEM((8, CH), jnp.int32),
        pltpu.VMEM((8, CH), jnp.int32),
        pltpu.VMEM((8, CH), F32),
        pltpu.VMEM((CH, D), F32),
        pltpu.VMEM((CH, D), F32),
        pltpu.VMEM((NPAD,), F32),
        pltpu.VMEM((ROWS_PER_TILE,), F32),
        pltpu.VMEM_SHARED((NPAD, D), F32),
        pltpu.SemaphoreType.DMA,
        pltpu.SemaphoreType.DMA,
    ]

    def body(hlh, srch, dsth, exh, denph, outph,
             src8, dst8, ex8, rows0, rows1,
             rden, dbuf, outsh, semG0, semG1):
        c = lax.axis_index("c")
        s = lax.axis_index("s")
        wid = s * 2 + c
        row0 = wid * nchunks
        rows = [rows0, rows1]
        semG = [semG0, semG1]

        pltpu.sync_copy(denph.at[0], rden)
        for piece in range(NPAD // ROWS_PER_TILE):
            po = piece * ROWS_PER_TILE
            pltpu.sync_copy(denph.at[1, pl.ds(po, ROWS_PER_TILE)], dbuf)

            def rd(i):
                sl = pl.ds(i * 16, 16)
                rden[pl.ds(po + i * 16, 16)] = 1.0 / (
                    rden[pl.ds(po + i * 16, 16)] + dbuf[sl] + 1e-16)

            pl.loop(0, ROWS_PER_TILE // 16)(rd)

        zeros16 = jnp.zeros((16,), F32)

        def zf(r):
            for v in range(8):
                rows0[r, pl.ds(v * 16, 16)] = zeros16

        pl.loop(0, CH)(zf)
        off = s * ROWS_PER_TILE
        for piece in range(ROWS_PER_TILE // CH):
            pltpu.sync_copy(rows0, outsh.at[pl.ds(off + piece * CH, CH)])
        plsc.subcore_barrier()

        def issue(j):
            pltpu.async_copy(hlh.at[src8.at[j]], rows[j & 1], semG[j & 1])

        def wait(b):
            pltpu.make_async_copy(hlh.at[src8.at[0]], rows[b], semG[b]).wait()

        def compute(j):
            b = j & 1

            def group(g):
                sl = pl.ds(g * 16, 16)
                dv = dst8[j, sl]
                w16 = ex8[j, sl] * plsc.load_gather(rden, [dv])
                for k in range(16):
                    e = g * 16 + k
                    w = w16[k]
                    for v in range(8):
                        s2 = pl.ds(v * 16, 16)
                        rows[b][e, s2] = rows[b][e, s2] * w

            pl.loop(0, CH // 16)(group)
            pltpu.sync_copy(rows[b], outsh.at[dst8.at[j]], add=True)

        def outer(oi):
            base_row = row0 + oi * 8
            pltpu.sync_copy(srch.at[pl.ds(base_row, 8)], src8)
            pltpu.sync_copy(dsth.at[pl.ds(base_row, 8)], dst8)
            pltpu.sync_copy(exh.at[pl.ds(base_row, 8)], ex8)
            issue(0)
            for j in range(8):
                if j + 1 < 8:
                    issue(j + 1)
                wait(j & 1)
                compute(j)

        pl.loop(0, nchunks // 8)(outer)
        plsc.subcore_barrier()

        for piece in range(ROWS_PER_TILE // CH):
            o = off + piece * CH
            pltpu.sync_copy(outsh.at[pl.ds(o, CH)], rows0)
            pltpu.sync_copy(rows0, outph.at[c, pl.ds(o, CH)])

    out_type = jax.ShapeDtypeStruct((2, NPAD, D), F32)
    return pl.kernel(body, out_type=out_type, mesh=mesh, scratch_types=scratch,
                     compiler_params=pltpu.CompilerParams(needs_layout_passes=False))


def _impl(x_product, x_customer, edge_index_pp, edge_index_pc, edge_label,
          Wl1, bl1, Wr1, br1, att1, b1,
          Wl2, bl2, Wr2, br2, We2, att2, b2,
          Wl3, bl3, Wr3, br3, We3, att3, b3,
          Wlin, blin):
    n_p = x_product.shape[0]
    epp = edge_index_pp.shape[1]
    epc = edge_index_pc.shape[1]

    xp = jnp.pad(x_product, ((0, NPAD - n_p), (0, 0)))
    xc = jnp.pad(x_customer, ((0, NPAD - x_customer.shape[0]), (0, 0)))

    e1 = epp + n_p
    e1p = _pad_edges(e1)
    loop = jnp.arange(n_p, dtype=jnp.int32)
    padv = jnp.full((e1p - e1,), PAD_IDX, jnp.int32)
    src1 = jnp.concatenate([edge_index_pp[0].astype(jnp.int32), loop, padv]).reshape(-1, CH)
    dst1 = jnp.concatenate([edge_index_pp[1].astype(jnp.int32), loop, padv]).reshape(-1, CH)

    e2p = _pad_edges(epc)
    padv2 = jnp.full((e2p - epc,), PAD_IDX, jnp.int32)
    src2 = jnp.concatenate([edge_index_pc[0].astype(jnp.int32), padv2]).reshape(-1, CH)
    dst2 = jnp.concatenate([edge_index_pc[1].astype(jnp.int32), padv2]).reshape(-1, CH)
    el2 = jnp.pad(edge_label, ((0, e2p - epc), (0, 0))).T

    pass_a1 = _make_pass_a(e1p, has_edge=False)
    pass_a2 = _make_pass_a(e2p, has_edge=True)
    pass_b1 = _make_pass_b(e1p)
    pass_b2 = _make_pass_b(e2p)

    hl1, hr1 = _mm2_same_x(xp, Wl1, bl1, Wr1, br1)
    ex1, den1 = pass_a1(hl1, hr1, src1, dst1, att1)
    outp1 = pass_b1(hl1, src1, dst1, ex1, den1)

    hl2, hr2 = _mm2_two_x(xp, Wl2, bl2, xc, Wr2, br2)
    ex2, den2 = pass_a2(hl2, hr2, src2, dst2, el2, We2, att2)
    outp2 = pass_b2(hl2, src2, dst2, ex2, den2)

    hl3 = _combine_mm(outp1[0], outp1[1], b1, Wl3, bl3)
    hr3 = _combine_mm(outp2[0], outp2[1], b2, Wr3, br3)

    ex3, den3 = pass_a2(hl3, hr3, src2, dst2, el2, We3, att3)
    outp3 = pass_b2(hl3, src2, dst2, ex3, den3)

    y = _combine_mm(outp3[0], outp3[1], b3, Wlin, blin)
    return y[:x_customer.shape[0]]


kernel = jax.jit(_impl)


# speedup vs baseline: 3.1628x; 3.1628x over previous
import jax
import jax.numpy as jnp
from jax import lax
from jax.experimental import pallas as pl
from jax.experimental.pallas import tpu as pltpu
from jax.experimental.pallas import tpu_sc as plsc

N = 10000
NPAD = 10240
D = 128
CH = 128
NW = 32
ROWS_PER_TILE = NPAD // 16
PAD_IDX = N

F32 = jnp.float32


def _pad_edges(ep):
    q = 8 * NW * CH
    return ((ep + q - 1) // q) * q


_BM = 512


def _mm2_same_x(x, Wl, bl, Wr, br):
    def body(x_ref, wl_ref, bl_ref, wr_ref, br_ref, o1_ref, o2_ref):
        xb = x_ref[...]
        o1_ref[...] = jnp.dot(xb, wl_ref[...], preferred_element_type=F32) + bl_ref[...]
        o2_ref[...] = jnp.dot(xb, wr_ref[...], preferred_element_type=F32) + br_ref[...]

    grid = (NPAD // _BM,)
    return pl.pallas_call(
        body,
        grid=grid,
        in_specs=[
            pl.BlockSpec((_BM, D), lambda i: (i, 0)),
            pl.BlockSpec((D, D), lambda i: (0, 0)),
            pl.BlockSpec((1, D), lambda i: (0, 0)),
            pl.BlockSpec((D, D), lambda i: (0, 0)),
            pl.BlockSpec((1, D), lambda i: (0, 0)),
        ],
        out_specs=[
            pl.BlockSpec((_BM, D), lambda i: (i, 0)),
            pl.BlockSpec((_BM, D), lambda i: (i, 0)),
        ],
        out_shape=[
            jax.ShapeDtypeStruct((NPAD, D), F32),
            jax.ShapeDtypeStruct((NPAD, D), F32),
        ],
    )(x, Wl, bl.reshape(1, D), Wr, br.reshape(1, D))


def _mm2_two_x(xl, Wl, bl, xr, Wr, br):
    def body(xl_ref, wl_ref, bl_ref, xr_ref, wr_ref, br_ref, o1_ref, o2_ref):
        o1_ref[...] = jnp.dot(xl_ref[...], wl_ref[...], preferred_element_type=F32) + bl_ref[...]
        o2_ref[...] = jnp.dot(xr_ref[...], wr_ref[...], preferred_element_type=F32) + br_ref[...]

    grid = (NPAD // _BM,)
    return pl.pallas_call(
        body,
        grid=grid,
        in_specs=[
            pl.BlockSpec((_BM, D), lambda i: (i, 0)),
            pl.BlockSpec((D, D), lambda i: (0, 0)),
            pl.BlockSpec((1, D), lambda i: (0, 0)),
            pl.BlockSpec((_BM, D), lambda i: (i, 0)),
            pl.BlockSpec((D, D), lambda i: (0, 0)),
            pl.BlockSpec((1, D), lambda i: (0, 0)),
        ],
        out_specs=[
            pl.BlockSpec((_BM, D), lambda i: (i, 0)),
            pl.BlockSpec((_BM, D), lambda i: (i, 0)),
        ],
        out_shape=[
            jax.ShapeDtypeStruct((NPAD, D), F32),
            jax.ShapeDtypeStruct((NPAD, D), F32),
        ],
    )(xl, Wl, bl.reshape(1, D), xr, Wr, br.reshape(1, D))


def _combine_mm(p0, p1, b, W, c):
    def body(p0_ref, p1_ref, b_ref, w_ref, c_ref, o_ref):
        y = jax.nn.relu(p0_ref[...] + p1_ref[...] + b_ref[...])
        o_ref[...] = jnp.dot(y, w_ref[...], preferred_element_type=F32) + c_ref[...]

    grid = (NPAD // _BM,)
    return pl.pallas_call(
        body,
        grid=grid,
        in_specs=[
            pl.BlockSpec((_BM, D), lambda i: (i, 0)),
            pl.BlockSpec((_BM, D), lambda i: (i, 0)),
            pl.BlockSpec((1, D), lambda i: (0, 0)),
            pl.BlockSpec((D, D), lambda i: (0, 0)),
            pl.BlockSpec((1, D), lambda i: (0, 0)),
        ],
        out_specs=pl.BlockSpec((_BM, D), lambda i: (i, 0)),
        out_shape=jax.ShapeDtypeStruct((NPAD, D), F32),
    )(p0, p1, b.reshape(1, D), W, c.reshape(1, D))


def _make_pass_a(ep_pad, has_edge):
    ept = ep_pad // NW
    nchunks = ept // CH
    mesh = plsc.VectorSubcoreMesh(core_axis_name="c", subcore_axis_name="s")

    scratch = [
        pltpu.VMEM((nchunks, CH), jnp.int32),
        pltpu.VMEM((nchunks, CH), jnp.int32),
        pltpu.VMEM((CH, D), F32),
        pltpu.VMEM((CH, D), F32),
        pltpu.VMEM((CH, D), F32),
        pltpu.VMEM((CH, D), F32),
        pltpu.VMEM((D,), F32),
        pltpu.VMEM((nchunks, CH), F32),
        pltpu.VMEM((ROWS_PER_TILE,), F32),
        pltpu.VMEM_SHARED((NPAD,), F32),
        pltpu.SemaphoreType.DMA,
        pltpu.SemaphoreType.DMA,
        pltpu.SemaphoreType.DMA,
        pltpu.SemaphoreType.DMA,
    ]
    if has_edge:
        scratch += [
            pltpu.VMEM((4, CH), F32),
            pltpu.VMEM((4, CH), F32),
            pltpu.VMEM((4, D), F32),
            pltpu.SemaphoreType.DMA,
            pltpu.SemaphoreType.DMA,
        ]

    def body(*refs):
        if has_edge:
            (hlh, hrh, srch, dsth, elh, weh, atth, exh, denph,
             src_all, dst_all, rl0, rl1, rr0, rr1, attv, exbig, zb, densh,
             semL0, semL1, semR0, semR1, elv0, elv1, wev, semE0, semE1) = refs
        else:
            (hlh, hrh, srch, dsth, atth, exh, denph,
             src_all, dst_all, rl0, rl1, rr0, rr1, attv, exbig, zb, densh,
             semL0, semL1, semR0, semR1) = refs

        c = lax.axis_index("c")
        s = lax.axis_index("s")
        wid = s * 2 + c
        row0 = wid * nchunks

        rl = [rl0, rl1]
        rr = [rr0, rr1]
        semL = [semL0, semL1]
        semR = [semR0, semR1]
        if has_edge:
            elv = [elv0, elv1]
            semE = [semE0, semE1]

        zeros16 = jnp.zeros((16,), F32)
        for i in range(ROWS_PER_TILE // 16):
            zb[pl.ds(i * 16, 16)] = zeros16
        pltpu.sync_copy(zb, densh.at[pl.ds(s * ROWS_PER_TILE, ROWS_PER_TILE)])
        pltpu.sync_copy(atth, attv)
        if has_edge:
            pltpu.sync_copy(weh, wev)
        pltpu.sync_copy(srch.at[pl.ds(row0, nchunks)], src_all)
        pltpu.sync_copy(dsth.at[pl.ds(row0, nchunks)], dst_all)
        plsc.subcore_barrier()

        def issue(ci, b):
            pltpu.async_copy(hlh.at[src_all.at[ci]], rl[b], semL[b])
            pltpu.async_copy(hrh.at[dst_all.at[ci]], rr[b], semR[b])
            if has_edge:
                for j in range(4):
                    pltpu.async_copy(elh.at[j, pl.ds((row0 + ci) * CH, CH)],
                                     elv[b].at[j], semE[b])

        def wait(b):
            pltpu.make_async_copy(hlh.at[src_all.at[0]], rl[b], semL[b]).wait()
            pltpu.make_async_copy(hrh.at[dst_all.at[0]], rr[b], semR[b]).wait()
            if has_edge:
                for j in range(4):
                    pltpu.make_async_copy(elh.at[j, pl.ds(0, CH)],
                                          elv[b].at[j], semE[b]).wait()

        attr = [attv[pl.ds(v * 16, 16)] for v in range(8)]
        if has_edge:
            wer = [[wev[j, pl.ds(v * 16, 16)] for v in range(8)] for j in range(4)]
        lane = lax.iota(jnp.int32, 16)
        masks = [lane == k for k in range(16)]

        def compute(ci, b):
            def group(g):
                if has_edge:
                    elg = [elv[b][j, pl.ds(g * 16, 16)] for j in range(4)]
                z = zeros16
                for k in range(16):
                    e = g * 16 + k
                    acc = zeros16
                    for v in range(8):
                        sl = pl.ds(v * 16, 16)
                        t = rl[b][e, sl] + rr[b][e, sl]
                        if has_edge:
                            t = (t + elg[0][k] * wer[0][v] + elg[1][k] * wer[1][v]
                                 + elg[2][k] * wer[2][v] + elg[3][k] * wer[3][v])
                        u = 0.6 * t + 0.4 * jnp.abs(t)
                        acc = acc + attr[v] * u
                    z = jnp.where(masks[k], jnp.sum(acc), z)
                exbig[ci, pl.ds(g * 16, 16)] = jnp.exp(z)

            pl.loop(0, CH // 16)(group)

        issue(0, 0)

        def step(ci):
            issue(ci + 1, 1)
            wait(0)
            compute(ci, 0)

            @pl.when(ci + 2 < nchunks)
            def _():
                issue(ci + 2, 0)

            wait(1)
            compute(ci + 1, 1)

        pl.loop(0, nchunks, step=2)(step)

        pltpu.sync_copy(exbig, exh.at[pl.ds(row0, nchunks)])
        nb = (nchunks // 8) * 8

        def scat8(ci):
            for j in range(8):
                pltpu.async_copy(exbig.at[ci + j], densh.at[dst_all.at[ci + j]],
                                 semL0, add=True)
            for j in range(8):
                pltpu.make_async_copy(exbig.at[0], densh.at[dst_all.at[0]],
                                      semL0).wait()

        pl.loop(0, nb, step=8)(scat8)
        for t in range(nchunks - nb):
            pltpu.sync_copy(exbig.at[nb + t], densh.at[dst_all.at[nb + t]],
                            add=True)
        plsc.subcore_barrier()

        off = s * ROWS_PER_TILE
        pltpu.sync_copy(densh.at[pl.ds(off, ROWS_PER_TILE)], zb)
        pltpu.sync_copy(zb, denph.at[c, pl.ds(off, ROWS_PER_TILE)])

    out_type = (
        jax.ShapeDtypeStruct((ep_pad // CH, CH), F32),
        jax.ShapeDtypeStruct((2, NPAD), F32),
    )
    return pl.kernel(body, out_type=out_type, mesh=mesh, scratch_types=scratch,
                     compiler_params=pltpu.CompilerParams(needs_layout_passes=False))


def _make_pass_b(ep_pad):
    ept = ep_pad // NW
    nchunks = ept // CH
    mesh = plsc.VectorSubcoreMesh(core_axis_name="c", subcore_axis_name="s")
    HALF = ROWS_PER_TILE // 2

    scratch = [
        pltpu.VMEM((8, CH), jnp.int32),
        pltpu.VMEM((8, CH), jnp.int32),
        pltpu.VMEM((8, CH), F32),
        pltpu.VMEM((CH, D), F32),
        pltpu.VMEM((CH, D), F32),
        pltpu.VMEM((NPAD,), F32),
        pltpu.VMEM((ROWS_PER_TILE,), F32),
        pltpu.VMEM_SHARED((NPAD, D), F32),
        pltpu.SemaphoreType.DMA,
        pltpu.SemaphoreType.DMA,
    ]

    def body(hlh, srch, dsth, exh, denph, outph,
             src8, dst8, ex8, rows0, rows1,
             rden, dbuf, outsh, semG0, semG1):
        c = lax.axis_index("c")
        s = lax.axis_index("s")
        wid = s * 2 + c
        row0 = wid * nchunks
        rows = [rows0, rows1]
        semG = [semG0, semG1]

        pltpu.sync_copy(denph.at[0], rden)
        for piece in range(NPAD // ROWS_PER_TILE):
            po = piece * ROWS_PER_TILE
            pltpu.sync_copy(denph.at[1, pl.ds(po, ROWS_PER_TILE)], dbuf)

            def rd(i):
                sl = pl.ds(i * 16, 16)
                rden[pl.ds(po + i * 16, 16)] = 1.0 / (
                    rden[pl.ds(po + i * 16, 16)] + dbuf[sl] + 1e-16)

            pl.loop(0, ROWS_PER_TILE // 16)(rd)

        zeros16 = jnp.zeros((16,), F32)

        def zf(r):
            for v in range(8):
                rows0[r, pl.ds(v * 16, 16)] = zeros16

        pl.loop(0, CH)(zf)
        off = s * ROWS_PER_TILE
        for piece in range(ROWS_PER_TILE // CH):
            pltpu.sync_copy(rows0, outsh.at[pl.ds(off + piece * CH, CH)])
        plsc.subcore_barrier()

        def issue(j):
            pltpu.async_copy(hlh.at[src8.at[j]], rows[j & 1], semG[j & 1])

        def wait(b):
            pltpu.make_async_copy(hlh.at[src8.at[0]], rows[b], semG[b]).wait()

        def compute(j):
            b = j & 1

            def group(g):
                sl = pl.ds(g * 16, 16)
                dv = dst8[j, sl]
                w16 = ex8[j, sl] * plsc.load_gather(rden, [dv])
                for k in range(16):
                    e = g * 16 + k
                    w = w16[k]
                    for v in range(8):
                        s2 = pl.ds(v * 16, 16)
                        rows[b][e, s2] = rows[b][e, s2] * w

            pl.loop(0, CH // 16)(group)
            pltpu.sync_copy(rows[b], outsh.at[dst8.at[j]], add=True)

        def outer(oi):
            base_row = row0 + oi * 8
            pltpu.sync_copy(srch.at[pl.ds(base_row, 8)], src8)
            pltpu.sync_copy(dsth.at[pl.ds(base_row, 8)], dst8)
            pltpu.sync_copy(exh.at[pl.ds(base_row, 8)], ex8)
            issue(0)
            for j in range(8):
                if j + 1 < 8:
                    issue(j + 1)
                wait(j & 1)
                compute(j)

        pl.loop(0, nchunks // 8)(outer)
        plsc.subcore_barrier()

        for piece in range(ROWS_PER_TILE // CH):
            o = off + piece * CH
            pltpu.sync_copy(outsh.at[pl.ds(o, CH)], rows0)
            pltpu.sync_copy(rows0, outph.at[c, pl.ds(o, CH)])

    out_type = jax.ShapeDtypeStruct((2, NPAD, D), F32)
    return pl.kernel(body, out_type=out_type, mesh=mesh, scratch_types=scratch,
                     compiler_params=pltpu.CompilerParams(needs_layout_passes=False))


def _impl(x_product, x_customer, edge_index_pp, edge_index_pc, edge_label,
          Wl1, bl1, Wr1, br1, att1, b1,
          Wl2, bl2, Wr2, br2, We2, att2, b2,
          Wl3, bl3, Wr3, br3, We3, att3, b3,
          Wlin, blin):
    n_p = x_product.shape[0]
    epp = edge_index_pp.shape[1]
    epc = edge_index_pc.shape[1]

    xp = jnp.pad(x_product, ((0, NPAD - n_p), (0, 0)))
    xc = jnp.pad(x_customer, ((0, NPAD - x_customer.shape[0]), (0, 0)))

    def _padv(n):
        return PAD_IDX + (jnp.arange(n, dtype=jnp.int32) % (NPAD - N))

    e1 = epp + n_p
    e1p = _pad_edges(e1)
    loop = jnp.arange(n_p, dtype=jnp.int32)
    padv = _padv(e1p - e1)
    src1 = jnp.concatenate([edge_index_pp[0].astype(jnp.int32), loop, padv]).reshape(-1, CH)
    dst1 = jnp.concatenate([edge_index_pp[1].astype(jnp.int32), loop, padv]).reshape(-1, CH)

    e2p = _pad_edges(epc)
    padv2 = _padv(e2p - epc)
    src2 = jnp.concatenate([edge_index_pc[0].astype(jnp.int32), padv2]).reshape(-1, CH)
    dst2 = jnp.concatenate([edge_index_pc[1].astype(jnp.int32), padv2]).reshape(-1, CH)
    el2 = jnp.pad(edge_label, ((0, e2p - epc), (0, 0))).T

    pass_a1 = _make_pass_a(e1p, has_edge=False)
    pass_a2 = _make_pass_a(e2p, has_edge=True)
    pass_b1 = _make_pass_b(e1p)
    pass_b2 = _make_pass_b(e2p)

    hl1, hr1 = _mm2_same_x(xp, Wl1, bl1, Wr1, br1)
    ex1, den1 = pass_a1(hl1, hr1, src1, dst1, att1)
    outp1 = pass_b1(hl1, src1, dst1, ex1, den1)

    hl2, hr2 = _mm2_two_x(xp, Wl2, bl2, xc, Wr2, br2)
    ex2, den2 = pass_a2(hl2, hr2, src2, dst2, el2, We2, att2)
    outp2 = pass_b2(hl2, src2, dst2, ex2, den2)

    hl3 = _combine_mm(outp1[0], outp1[1], b1, Wl3, bl3)
    hr3 = _combine_mm(outp2[0], outp2[1], b2, Wr3, br3)

    ex3, den3 = pass_a2(hl3, hr3, src2, dst2, el2, We3, att3)
    outp3 = pass_b2(hl3, src2, dst2, ex3, den3)

    y = _combine_mm(outp3[0], outp3[1], b3, Wlin, blin)
    return y[:x_customer.shape[0]]


kernel = jax.jit(_impl)
